# baseline (speedup 1.0000x reference)
import functools

import jax
import jax.numpy as jnp
from jax import lax
from jax.experimental import pallas as pl
from jax.experimental.pallas import tpu as pltpu
from jax.experimental.pallas import tpu_sc as plsc

NUM_CODES = 1024
CODE_DIM = 64
N_TOK = 36864
BETA = 0.25

BLK = 2304
GRID = N_TOK // BLK

NC = 2
NS = 16
NW = NC * NS
ROWS_PER_W = N_TOK // NW
IDX_CHUNK = 128
CHUNKS_PER_W = ROWS_PER_W // IDX_CHUNK


def _dist_argmin_body(z_ref, cb_ref, idx_ref, loss_ref, cbsq_ref, iota_ref,
                      cbt2_ref):
    @pl.when(pl.program_id(0) == 0)
    def _init():
        cb = cb_ref[...]
        cbsq_ref[...] = jnp.sum(cb * cb, axis=1)[None, :]
        cbt2_ref[...] = (cb + cb).T
        iota_ref[...] = lax.broadcasted_iota(
            jnp.int32, (1, NUM_CODES), 1).astype(jnp.float32)
        loss_ref[...] = jnp.zeros_like(loss_ref)

    z = z_ref[...]
    mm2 = jnp.dot(z, cbt2_ref[...], preferred_element_type=jnp.float32)
    z_sq = jnp.sum(z * z, axis=1, keepdims=True)
    LW = 128
    run_min = None
    for c in range(NUM_CODES // LW):
        cs = slice(c * LW, (c + 1) * LW)
        dc = (z_sq + cbsq_ref[...][:, cs]) - mm2[:, cs]
        ic = iota_ref[...][:, cs]
        if run_min is None:
            run_min, run_idx = dc, jnp.broadcast_to(ic, dc.shape)
        else:
            better = dc < run_min
            run_idx = jnp.where(better, ic, run_idx)
            run_min = jnp.minimum(run_min, dc)
    m = jnp.min(run_min, axis=1, keepdims=True)
    idx = jnp.min(jnp.where(run_min == m, run_idx, float(NUM_CODES)),
                  axis=1, keepdims=True)
    idx_ref[...] = idx.astype(jnp.int32).reshape(1, BLK // 128, 128)
    loss_ref[...] += jnp.sum(m, axis=0, keepdims=True)


def _make_dist_call(n_tok):
    grid = n_tok // BLK
    return pl.pallas_call(
        _dist_argmin_body,
        grid=(grid,),
        in_specs=[
            pl.BlockSpec((BLK, CODE_DIM), lambda i: (i, 0)),
            pl.BlockSpec((NUM_CODES, CODE_DIM), lambda i: (0, 0)),
        ],
        out_specs=[
            pl.BlockSpec((1, BLK // 128, 128), lambda i: (i, 0, 0)),
            pl.BlockSpec((1, 1), lambda i: (0, 0)),
        ],
        out_shape=[
            jax.ShapeDtypeStruct((grid, BLK // 128, 128), jnp.int32),
            jax.ShapeDtypeStruct((1, 1), jnp.float32),
        ],
        scratch_shapes=[pltpu.VMEM((1, NUM_CODES), jnp.float32),
                        pltpu.VMEM((1, NUM_CODES), jnp.float32),
                        pltpu.VMEM((CODE_DIM, NUM_CODES), jnp.float32)],
    )


def _make_sc_gather_body(rows_per_w, idx_chunk):
    n_chunks = rows_per_w // idx_chunk

    def _sc_gather_body(cb_hbm, idx_hbm, out_hbm, idx_v, rows_v, sem):
        wid = lax.axis_index("s") * NC + lax.axis_index("c")
        pltpu.sync_copy(idx_hbm.at[pl.ds(wid * rows_per_w, rows_per_w)], idx_v)
        copies = [
            pltpu.async_copy(
                cb_hbm.at[idx_v.at[pl.ds(j * idx_chunk, idx_chunk)]],
                rows_v.at[pl.ds(j * idx_chunk, idx_chunk)],
                sem,
            )
            for j in range(n_chunks)
        ]
        for c in copies:
            c.wait()
        pltpu.sync_copy(rows_v, out_hbm.at[pl.ds(wid * rows_per_w, rows_per_w)])

    return _sc_gather_body


@functools.cache
def _sc_gather(n_tok, idx_chunk):
    rows_per_w = n_tok // NW
    return pl.kernel(
        _make_sc_gather_body(rows_per_w, idx_chunk),
        out_type=jax.ShapeDtypeStruct((n_tok, CODE_DIM), jnp.float32),
        mesh=plsc.VectorSubcoreMesh(core_axis_name="c", subcore_axis_name="s"),
        compiler_params=pltpu.CompilerParams(use_tc_tiling_on_sc=False),
        scratch_types=[
            pltpu.VMEM((rows_per_w,), jnp.int32),
            pltpu.VMEM((rows_per_w, CODE_DIM), jnp.float32),
            pltpu.SemaphoreType.DMA,
        ],
    )


def kernel(z, codebook):
    half = N_TOK // 2
    dist = _make_dist_call(half)
    idx_a, loss_a = dist(z[:half], codebook)
    ind_a = idx_a.reshape(half)
    zq_a = _sc_gather(half, 96)(codebook, ind_a)
    idx_b, loss_b = dist(z[half:], codebook)
    ind_b = idx_b.reshape(half)
    zq_b = _sc_gather(half, 96)(codebook, ind_b)
    z_q = jnp.concatenate([zq_a, zq_b], axis=0)
    indices = jnp.concatenate([ind_a, ind_b], axis=0)
    commit_loss = (loss_a[0, 0] + loss_b[0, 0]) * (BETA / (N_TOK * CODE_DIM))
    return (z_q, indices, commit_loss)


# speedup vs baseline: 1.2267x; 1.2267x over previous
import functools

import jax
import jax.numpy as jnp
from jax import lax
from jax.experimental import pallas as pl
from jax.experimental.pallas import tpu as pltpu
from jax.experimental.pallas import tpu_sc as plsc

NUM_CODES = 1024
CODE_DIM = 64
N_TOK = 36864
BETA = 0.25

BLK = 4096
GRID = N_TOK // BLK

NC = 2
NS = 16
NW = NC * NS
ROWS_PER_W = N_TOK // NW
IDX_CHUNK = 128
CHUNKS_PER_W = ROWS_PER_W // IDX_CHUNK


def _dist_argmin_body(z_ref, cb_ref, idx_ref, loss_ref, cbsq_ref, iota_ref,
                      cbt2_ref):
    @pl.when(pl.program_id(0) == 0)
    def _init():
        cb = cb_ref[...]
        cbsq_ref[...] = jnp.sum(cb * cb, axis=1)[None, :]
        cbt2_ref[...] = (cb + cb).T
        iota_ref[...] = lax.broadcasted_iota(
            jnp.int32, (1, NUM_CODES), 1).astype(jnp.float32)
        loss_ref[...] = jnp.zeros_like(loss_ref)

    z = z_ref[...]
    mm2 = jnp.dot(z, cbt2_ref[...], preferred_element_type=jnp.float32)
    z_sq = jnp.sum(z * z, axis=1, keepdims=True)
    LW = 128
    run_min = None
    for c in range(NUM_CODES // LW):
        cs = slice(c * LW, (c + 1) * LW)
        dc = (z_sq + cbsq_ref[...][:, cs]) - mm2[:, cs]
        ic = iota_ref[...][:, cs]
        if run_min is None:
            run_min, run_idx = dc, jnp.broadcast_to(ic, dc.shape)
        else:
            better = dc < run_min
            run_idx = jnp.where(better, ic, run_idx)
            run_min = jnp.minimum(run_min, dc)
    m = jnp.min(run_min, axis=1, keepdims=True)
    idx = jnp.min(jnp.where(run_min == m, run_idx, float(NUM_CODES)),
                  axis=1, keepdims=True)
    idx_ref[...] = idx.astype(jnp.int32).reshape(1, BLK // 128, 128)
    loss_ref[...] += jnp.sum(m, axis=0, keepdims=True)


_dist_argmin_call = pl.pallas_call(
    _dist_argmin_body,
    grid=(GRID,),
    in_specs=[
        pl.BlockSpec((BLK, CODE_DIM), lambda i: (i, 0)),
        pl.BlockSpec((NUM_CODES, CODE_DIM), lambda i: (0, 0)),
    ],
    out_specs=[
        pl.BlockSpec((1, BLK // 128, 128), lambda i: (i, 0, 0)),
        pl.BlockSpec((1, 1), lambda i: (0, 0)),
    ],
    out_shape=[
        jax.ShapeDtypeStruct((GRID, BLK // 128, 128), jnp.int32),
        jax.ShapeDtypeStruct((1, 1), jnp.float32),
    ],
    scratch_shapes=[pltpu.VMEM((1, NUM_CODES), jnp.float32),
                    pltpu.VMEM((1, NUM_CODES), jnp.float32),
                    pltpu.VMEM((CODE_DIM, NUM_CODES), jnp.float32)],
)


def _sc_gather_body(cb_hbm, idx_hbm, out_hbm, idx_v, rows_v, sem):
    wid = lax.axis_index("s") * NC + lax.axis_index("c")
    pltpu.sync_copy(idx_hbm.at[pl.ds(wid * ROWS_PER_W, ROWS_PER_W)], idx_v)
    copies = [
        pltpu.async_copy(
            cb_hbm.at[idx_v.at[pl.ds(j * IDX_CHUNK, IDX_CHUNK)]],
            rows_v.at[pl.ds(j * IDX_CHUNK, IDX_CHUNK)],
            sem,
        )
        for j in range(CHUNKS_PER_W)
    ]
    for c in copies:
        c.wait()
    pltpu.sync_copy(rows_v, out_hbm.at[pl.ds(wid * ROWS_PER_W, ROWS_PER_W)])


@functools.cache
def _sc_gather():
    return pl.kernel(
        _sc_gather_body,
        out_type=jax.ShapeDtypeStruct((N_TOK, CODE_DIM), jnp.float32),
        mesh=plsc.VectorSubcoreMesh(core_axis_name="c", subcore_axis_name="s"),
        compiler_params=pltpu.CompilerParams(use_tc_tiling_on_sc=False),
        scratch_types=[
            pltpu.VMEM((ROWS_PER_W,), jnp.int32),
            pltpu.VMEM((ROWS_PER_W, CODE_DIM), jnp.float32),
            pltpu.SemaphoreType.DMA,
        ],
    )


def kernel(z, codebook):
    idx_rows, loss_sum = _dist_argmin_call(z, codebook)
    indices = idx_rows.reshape(N_TOK)
    z_q = _sc_gather()(codebook, indices)
    commit_loss = loss_sum[0, 0] * (BETA / (N_TOK * CODE_DIM))
    return (z_q, indices, commit_loss)
